# baseline (speedup 1.0000x reference)
import functools

import jax
import jax.numpy as jnp
import numpy as np
from jax import lax
from jax.experimental import pallas as pl
from jax.experimental.pallas import tpu as pltpu
from jax.experimental.pallas import tpu_sc as plsc

_THRESH = np.float32(0.1)
_SENTINEL = np.float32(-1.0)


def _make_sc_stats(B, H, W):
    HALF_ROWS = H // 2
    CHUNK_ROWS = 8
    n_chunks = HALF_ROWS // CHUNK_ROWS
    UNROLL = 4
    n_iter = CHUNK_ROWS * W // (16 * UNROLL)
    grp_per_row = W // 16
    assert HALF_ROWS % CHUNK_ROWS == 0 and grp_per_row == 32

    mesh = plsc.VectorSubcoreMesh(core_axis_name="c", subcore_axis_name="s")

    @functools.partial(
        pl.kernel,
        out_type=jax.ShapeDtypeStruct((2 * B, 8, 128), jnp.float32),
        mesh=mesh,
        scratch_types=[pltpu.VMEM((6, CHUNK_ROWS, W), jnp.float32)
                       for _ in range(5)]
        + [pltpu.VMEM((8, 128), jnp.float32), pltpu.SemaphoreType.DMA],
        compiler_params=pltpu.CompilerParams(use_tc_tiling_on_sc=True),
    )
    def sc_stats(gr_h, ga_h, pr_h, pa_h, cf_h, out_h,
                 vgr, vga, vpr, vpa, vcf, vacc, sem):
        sample = lax.axis_index("s")
        half = lax.axis_index("c")
        row0 = half * HALF_ROWS
        hbm = (gr_h, ga_h, pr_h, pa_h, cf_h)
        bufs = (vgr, vga, vpr, vpa, vcf)

        NBUF = 6

        def start(j):
            rs = pl.ds(row0 + j * CHUNK_ROWS, CHUNK_ROWS)
            return [pltpu.async_copy(h.at[sample, rs, :], v.at[j % NBUF], sem)
                    for h, v in zip(hbm, bufs)]

        zero = jnp.zeros((16,), jnp.float32)
        accs = (zero, zero, zero, zero, zero, zero)
        pending = [start(j) for j in range(NBUF - 1)]
        for j in range(n_chunks):
            for cp in pending.pop(0):
                cp.wait()
            if j + NBUF - 1 < n_chunks:
                pending.append(start(j + NBUF - 1))
            slot = j % NBUF

            def body(i, acc, slot=slot):
                npr, psr, tsr, npa, psa, tsa = acc
                r = lax.shift_right_logical(i, 3)
                c0 = lax.shift_left(lax.bitwise_and(i, 7), 6)
                for u in range(UNROLL):
                    sl = pl.ds(pl.multiple_of(c0 + u * 16, 16), 16)
                    g_r = vgr[slot, r, sl]
                    g_a = vga[slot, r, sl]
                    p_r = vpr[slot, r, sl]
                    p_a = vpa[slot, r, sl]
                    c = vcf[slot, r, sl]
                    d_r = p_r - g_r
                    l_r = d_r * d_r * c
                    d_a = p_a - g_a
                    l_a = d_a * d_a * c
                    m_r = g_r >= _THRESH
                    m_a = g_a >= _THRESH
                    npr = npr + jnp.where(m_r, np.float32(1.0), np.float32(0.0))
                    psr = psr + jnp.where(m_r, l_r, np.float32(0.0))
                    tsr = tsr + l_r
                    npa = npa + jnp.where(m_a, np.float32(1.0), np.float32(0.0))
                    psa = psa + jnp.where(m_a, l_a, np.float32(0.0))
                    tsa = tsa + l_a
                return (npr, psr, tsr, npa, psa, tsa)

            accs = lax.fori_loop(0, n_iter, body, accs)

        for q in range(6):
            vacc[q, pl.ds(0, 16)] = accs[q]
        wid = sample * 2 + half
        pltpu.sync_copy(vacc, out_h.at[wid])

    return sc_stats


def _fb_body(P, gt_ref, pd_ref, cf_ref, out_ref):
    g = gt_ref[0, 0]
    p = pd_ref[0, 0]
    c = cf_ref[0]
    d = p - g
    l = d * d * c
    m = g >= _THRESH
    one = np.float32(1.0)
    zero = np.float32(0.0)
    npos = jnp.sum(jnp.where(m, one, zero))
    possum = jnp.sum(jnp.where(m, l, zero))
    negsum = jnp.sum(jnp.where(m, zero, l))
    nneg = np.float32(P) - npos
    k = np.float32(3.0) * npos
    neg_vals = jnp.where(m, _SENTINEL, l)

    def topsum(vals, kk):
        def bis(_, lohi):
            lo, hi = lohi
            mid = (lo + hi) * np.float32(0.5)
            cnt = jnp.sum(jnp.where(vals > mid, one, zero))
            take = cnt >= kk
            return (jnp.where(take, mid, lo), jnp.where(take, hi, mid))

        lo, hi = lax.fori_loop(
            0, 220, bis, (np.float32(-2.0), np.float32(2.0)))
        vk = hi
        gt_m = vals > vk
        cnt = jnp.sum(jnp.where(gt_m, one, zero))
        s = jnp.sum(jnp.where(gt_m, vals, zero))
        return s + (kk - cnt) * vk

    k_eff = jnp.clip(k, one, np.float32(P))
    topk_sum = topsum(neg_vals, k_eff)
    top500_sum = topsum(l, np.float32(500.0))

    pos_loss = possum / jnp.maximum(npos, one)
    neg_mean_all = negsum / jnp.maximum(nneg, one)
    topk_mean = topk_sum / jnp.maximum(k, one)
    negative_loss = jnp.where(nneg < k, neg_mean_all, topk_mean)
    contrib = jnp.where(
        npos > zero, pos_loss + negative_loss, top500_sum / np.float32(500.0))
    out_ref[0, 0, 0, :] = jnp.full((128,), contrib, jnp.float32)


def _make_fallback(B, H, W, interpret=False):
    P = H * W

    def fallback(gt_r, gt_a, p_r, p_a, cf):
        gt = jnp.stack([gt_r, gt_a])
        pd = jnp.stack([p_r, p_a])
        out = pl.pallas_call(
            functools.partial(_fb_body, P),
            grid=(2, B),
            in_specs=[
                pl.BlockSpec((1, 1, H, W), lambda t, s: (t, s, 0, 0)),
                pl.BlockSpec((1, 1, H, W), lambda t, s: (t, s, 0, 0)),
                pl.BlockSpec((1, H, W), lambda t, s: (s, 0, 0)),
            ],
            out_specs=pl.BlockSpec((1, 1, 1, 128), lambda t, s: (t, s, 0, 0)),
            out_shape=jax.ShapeDtypeStruct((2, B, 1, 128), jnp.float32),
            interpret=interpret,
        )(gt, pd, cf)
        return jnp.sum(out[:, :, 0, 0]) / np.float32(B)

    return fallback


def kernel(gt_region_scores, gt_affinity_scores, pred_region_scores,
           pred_affinity_scores, confidence_mask):
    B, H, W = gt_region_scores.shape
    P = H * W

    sc_stats = _make_sc_stats(B, H, W)
    stats = sc_stats(
        gt_region_scores,
        gt_affinity_scores,
        pred_region_scores,
        pred_affinity_scores,
        confidence_mask,
    )

    st = stats.reshape(B, 2, 8, 128)[:, :, :6, :16].sum(axis=(1, 3))
    one = np.float32(1.0)
    cheap_total = np.float32(0.0)
    need = False
    for t in range(2):
        npos = st[:, 3 * t + 0]
        possum = st[:, 3 * t + 1]
        negsum = st[:, 3 * t + 2] - possum
        nneg = np.float32(P) - npos
        k = np.float32(3.0) * npos
        need = jnp.logical_or(need, jnp.any(
            jnp.logical_or(npos < np.float32(0.5), nneg >= k)))
        contrib = possum / jnp.maximum(npos, one) + negsum / jnp.maximum(nneg, one)
        cheap_total = cheap_total + jnp.sum(contrib)
    cheap = cheap_total / np.float32(B)

    fallback = _make_fallback(B, H, W)
    return lax.cond(
        need,
        lambda: fallback(gt_region_scores, gt_affinity_scores,
                         pred_region_scores, pred_affinity_scores,
                         confidence_mask),
        lambda: cheap,
    )


# speedup vs baseline: 1.1533x; 1.1533x over previous
import functools

import jax
import jax.numpy as jnp
import numpy as np
from jax import lax
from jax.experimental import pallas as pl
from jax.experimental.pallas import tpu as pltpu
from jax.experimental.pallas import tpu_sc as plsc

_THRESH = np.float32(0.1)
_SENTINEL = np.float32(-1.0)


def _make_sc_stats(n_sc, H, W):
    SUB_PER_SAMPLE = 32 // n_sc
    SUB_ROWS = H // SUB_PER_SAMPLE
    CHUNK_ROWS = 16
    n_chunks = SUB_ROWS // CHUNK_ROWS
    UNROLL = 4
    n_iter = CHUNK_ROWS * W // (16 * UNROLL)
    grp_per_row = W // 16
    assert SUB_ROWS % CHUNK_ROWS == 0 and grp_per_row == 32
    assert 32 % n_sc == 0

    mesh = plsc.VectorSubcoreMesh(core_axis_name="c", subcore_axis_name="s")

    @functools.partial(
        pl.kernel,
        out_type=jax.ShapeDtypeStruct((32, 8, 128), jnp.float32),
        mesh=mesh,
        scratch_types=[pltpu.VMEM((3, CHUNK_ROWS, W), jnp.float32)
                       for _ in range(5)]
        + [pltpu.VMEM((8, 128), jnp.float32), pltpu.SemaphoreType.DMA],
        compiler_params=pltpu.CompilerParams(use_tc_tiling_on_sc=True),
    )
    def sc_stats(gr_h, ga_h, pr_h, pa_h, cf_h, out_h,
                 vgr, vga, vpr, vpa, vcf, vacc, sem):
        wid = lax.axis_index("s") * 2 + lax.axis_index("c")
        sample = lax.div(wid, SUB_PER_SAMPLE)
        row0 = lax.rem(wid, SUB_PER_SAMPLE) * SUB_ROWS
        hbm = (gr_h, ga_h, pr_h, pa_h, cf_h)
        bufs = (vgr, vga, vpr, vpa, vcf)

        NBUF = 3

        def start(j):
            rs = pl.ds(row0 + j * CHUNK_ROWS, CHUNK_ROWS)
            return [pltpu.async_copy(h.at[sample, rs, :], v.at[j % NBUF], sem)
                    for h, v in zip(hbm, bufs)]

        zero = jnp.zeros((16,), jnp.float32)
        accs = (zero, zero, zero, zero, zero, zero)
        pending = [start(j) for j in range(NBUF - 1)]
        for j in range(n_chunks):
            for cp in pending.pop(0):
                cp.wait()
            if j + NBUF - 1 < n_chunks:
                pending.append(start(j + NBUF - 1))
            slot = j % NBUF

            def body(i, acc, slot=slot):
                npr, psr, tsr, npa, psa, tsa = acc
                r = lax.shift_right_logical(i, 3)
                c0 = lax.shift_left(lax.bitwise_and(i, 7), 6)
                for u in range(UNROLL):
                    sl = pl.ds(pl.multiple_of(c0 + u * 16, 16), 16)
                    g_r = vgr[slot, r, sl]
                    g_a = vga[slot, r, sl]
                    p_r = vpr[slot, r, sl]
                    p_a = vpa[slot, r, sl]
                    c = vcf[slot, r, sl]
                    d_r = p_r - g_r
                    l_r = d_r * d_r * c
                    d_a = p_a - g_a
                    l_a = d_a * d_a * c
                    m_r = g_r >= _THRESH
                    m_a = g_a >= _THRESH
                    npr = npr + jnp.where(m_r, np.float32(1.0), np.float32(0.0))
                    psr = psr + jnp.where(m_r, l_r, np.float32(0.0))
                    tsr = tsr + l_r
                    npa = npa + jnp.where(m_a, np.float32(1.0), np.float32(0.0))
                    psa = psa + jnp.where(m_a, l_a, np.float32(0.0))
                    tsa = tsa + l_a
                return (npr, psr, tsr, npa, psa, tsa)

            accs = lax.fori_loop(0, n_iter, body, accs)

        for q in range(6):
            vacc[q, pl.ds(0, 16)] = accs[q]
        pltpu.sync_copy(vacc, out_h.at[wid])

    return sc_stats


def _make_tc_stats(first, n, H, W):
    one = np.float32(1.0)
    zero = np.float32(0.0)

    def body(gr, ga, pr, pa, cf, out_ref):
        c = cf[0]
        rows = []
        for g, p in ((gr[0], pr[0]), (ga[0], pa[0])):
            d = p - g
            l = d * d * c
            m = g >= _THRESH
            rows.append(jnp.sum(jnp.where(m, one, zero)))
            rows.append(jnp.sum(jnp.where(m, l, zero)))
            rows.append(jnp.sum(l))
        block = jnp.concatenate(
            [jnp.full((1, 128), v, jnp.float32) for v in rows]
            + [jnp.zeros((2, 128), jnp.float32)], axis=0)
        out_ref[0] = block

    def run(gt_r, gt_a, p_r, p_a, cf):
        return pl.pallas_call(
            body,
            grid=(n,),
            in_specs=[pl.BlockSpec((1, H, W), lambda i: (i + first, 0, 0))
                      for _ in range(5)],
            out_specs=pl.BlockSpec((1, 8, 128), lambda i: (i, 0, 0)),
            out_shape=jax.ShapeDtypeStruct((n, 8, 128), jnp.float32),
        )(gt_r, gt_a, p_r, p_a, cf)

    return run


def _fb_body(P, gt_ref, pd_ref, cf_ref, out_ref):
    g = gt_ref[0, 0]
    p = pd_ref[0, 0]
    c = cf_ref[0]
    d = p - g
    l = d * d * c
    m = g >= _THRESH
    one = np.float32(1.0)
    zero = np.float32(0.0)
    npos = jnp.sum(jnp.where(m, one, zero))
    possum = jnp.sum(jnp.where(m, l, zero))
    negsum = jnp.sum(jnp.where(m, zero, l))
    nneg = np.float32(P) - npos
    k = np.float32(3.0) * npos
    neg_vals = jnp.where(m, _SENTINEL, l)

    def topsum(vals, kk):
        def bis(_, lohi):
            lo, hi = lohi
            mid = (lo + hi) * np.float32(0.5)
            cnt = jnp.sum(jnp.where(vals > mid, one, zero))
            take = cnt >= kk
            return (jnp.where(take, mid, lo), jnp.where(take, hi, mid))

        lo, hi = lax.fori_loop(
            0, 220, bis, (np.float32(-2.0), np.float32(2.0)))
        vk = hi
        gt_m = vals > vk
        cnt = jnp.sum(jnp.where(gt_m, one, zero))
        s = jnp.sum(jnp.where(gt_m, vals, zero))
        return s + (kk - cnt) * vk

    k_eff = jnp.clip(k, one, np.float32(P))
    topk_sum = topsum(neg_vals, k_eff)
    top500_sum = topsum(l, np.float32(500.0))

    pos_loss = possum / jnp.maximum(npos, one)
    neg_mean_all = negsum / jnp.maximum(nneg, one)
    topk_mean = topk_sum / jnp.maximum(k, one)
    negative_loss = jnp.where(nneg < k, neg_mean_all, topk_mean)
    contrib = jnp.where(
        npos > zero, pos_loss + negative_loss, top500_sum / np.float32(500.0))
    out_ref[0, 0, 0, :] = jnp.full((128,), contrib, jnp.float32)


def _make_fallback(B, H, W, interpret=False):
    P = H * W

    def fallback(gt_r, gt_a, p_r, p_a, cf):
        gt = jnp.stack([gt_r, gt_a])
        pd = jnp.stack([p_r, p_a])
        out = pl.pallas_call(
            functools.partial(_fb_body, P),
            grid=(2, B),
            in_specs=[
                pl.BlockSpec((1, 1, H, W), lambda t, s: (t, s, 0, 0)),
                pl.BlockSpec((1, 1, H, W), lambda t, s: (t, s, 0, 0)),
                pl.BlockSpec((1, H, W), lambda t, s: (s, 0, 0)),
            ],
            out_specs=pl.BlockSpec((1, 1, 1, 128), lambda t, s: (t, s, 0, 0)),
            out_shape=jax.ShapeDtypeStruct((2, B, 1, 128), jnp.float32),
            interpret=interpret,
        )(gt, pd, cf)
        return jnp.sum(out[:, :, 0, 0]) / np.float32(B)

    return fallback


def kernel(gt_region_scores, gt_affinity_scores, pred_region_scores,
           pred_affinity_scores, confidence_mask):
    B, H, W = gt_region_scores.shape
    P = H * W

    n_sc = B // 2
    sc_stats = _make_sc_stats(n_sc, H, W)
    tc_stats = _make_tc_stats(n_sc, B - n_sc, H, W)
    args = (gt_region_scores, gt_affinity_scores, pred_region_scores,
            pred_affinity_scores, confidence_mask)
    sc_out = sc_stats(*args)
    tc_out = tc_stats(*args)

    sub = 32 // n_sc
    st_sc = sc_out.reshape(n_sc, sub, 8, 128)[:, :, :6, :16].sum(axis=(1, 3))
    st_tc = tc_out[:, :6, 0]
    st = jnp.concatenate([st_sc, st_tc], axis=0)
    one = np.float32(1.0)
    cheap_total = np.float32(0.0)
    need = False
    for t in range(2):
        npos = st[:, 3 * t + 0]
        possum = st[:, 3 * t + 1]
        negsum = st[:, 3 * t + 2] - possum
        nneg = np.float32(P) - npos
        k = np.float32(3.0) * npos
        need = jnp.logical_or(need, jnp.any(
            jnp.logical_or(npos < np.float32(0.5), nneg >= k)))
        contrib = possum / jnp.maximum(npos, one) + negsum / jnp.maximum(nneg, one)
        cheap_total = cheap_total + jnp.sum(contrib)
    cheap = cheap_total / np.float32(B)

    fallback = _make_fallback(B, H, W)
    return lax.cond(
        need,
        lambda: fallback(gt_region_scores, gt_affinity_scores,
                         pred_region_scores, pred_affinity_scores,
                         confidence_mask),
        lambda: cheap,
    )


# speedup vs baseline: 1.1735x; 1.0176x over previous
import functools

import jax
import jax.numpy as jnp
import numpy as np
from jax import lax
from jax.experimental import pallas as pl
from jax.experimental.pallas import tpu as pltpu
from jax.experimental.pallas import tpu_sc as plsc

_THRESH = np.float32(0.1)
_SENTINEL = np.float32(-1.0)


def _make_sc_stats(n_sc, H, W):
    SUB_PER_SAMPLE = 32 // n_sc
    SUB_ROWS = H // SUB_PER_SAMPLE
    CHUNK_ROWS = 16
    n_chunks = SUB_ROWS // CHUNK_ROWS
    UNROLL = 4
    n_iter = CHUNK_ROWS * W // (16 * UNROLL)
    grp_per_row = W // 16
    assert SUB_ROWS % CHUNK_ROWS == 0 and grp_per_row == 32
    assert 32 % n_sc == 0

    mesh = plsc.VectorSubcoreMesh(core_axis_name="c", subcore_axis_name="s")

    @functools.partial(
        pl.kernel,
        out_type=jax.ShapeDtypeStruct((32, 8, 128), jnp.float32),
        mesh=mesh,
        scratch_types=[pltpu.VMEM((3, CHUNK_ROWS, W), jnp.float32)
                       for _ in range(5)]
        + [pltpu.VMEM((8, 128), jnp.float32), pltpu.SemaphoreType.DMA],
        compiler_params=pltpu.CompilerParams(use_tc_tiling_on_sc=True),
    )
    def sc_stats(gr_h, ga_h, pr_h, pa_h, cf_h, out_h,
                 vgr, vga, vpr, vpa, vcf, vacc, sem):
        wid = lax.axis_index("s") * 2 + lax.axis_index("c")
        sample = lax.div(wid, SUB_PER_SAMPLE)
        row0 = lax.rem(wid, SUB_PER_SAMPLE) * SUB_ROWS
        hbm = (gr_h, ga_h, pr_h, pa_h, cf_h)
        bufs = (vgr, vga, vpr, vpa, vcf)

        NBUF = 3

        def start(j):
            rs = pl.ds(row0 + j * CHUNK_ROWS, CHUNK_ROWS)
            return [pltpu.async_copy(h.at[sample, rs, :], v.at[j % NBUF], sem)
                    for h, v in zip(hbm, bufs)]

        zero = jnp.zeros((16,), jnp.float32)
        accs = (zero, zero, zero, zero, zero, zero)
        pending = [start(j) for j in range(NBUF - 1)]
        for j in range(n_chunks):
            for cp in pending.pop(0):
                cp.wait()
            if j + NBUF - 1 < n_chunks:
                pending.append(start(j + NBUF - 1))
            slot = j % NBUF

            def body(i, acc, slot=slot):
                npr, psr, tsr, npa, psa, tsa = acc
                r = lax.shift_right_logical(i, 3)
                c0 = lax.shift_left(lax.bitwise_and(i, 7), 6)
                for u in range(UNROLL):
                    sl = pl.ds(pl.multiple_of(c0 + u * 16, 16), 16)
                    g_r = vgr[slot, r, sl]
                    g_a = vga[slot, r, sl]
                    p_r = vpr[slot, r, sl]
                    p_a = vpa[slot, r, sl]
                    c = vcf[slot, r, sl]
                    d_r = p_r - g_r
                    l_r = d_r * d_r * c
                    d_a = p_a - g_a
                    l_a = d_a * d_a * c
                    m_r = g_r >= _THRESH
                    m_a = g_a >= _THRESH
                    npr = npr + jnp.where(m_r, np.float32(1.0), np.float32(0.0))
                    psr = psr + jnp.where(m_r, l_r, np.float32(0.0))
                    tsr = tsr + l_r
                    npa = npa + jnp.where(m_a, np.float32(1.0), np.float32(0.0))
                    psa = psa + jnp.where(m_a, l_a, np.float32(0.0))
                    tsa = tsa + l_a
                return (npr, psr, tsr, npa, psa, tsa)

            accs = lax.fori_loop(0, n_iter, body, accs)

        for q in range(6):
            vacc[q, pl.ds(0, 16)] = accs[q]
        pltpu.sync_copy(vacc, out_h.at[wid])

    return sc_stats


def _make_tc_stats(first, n, H, W):
    one = np.float32(1.0)
    zero = np.float32(0.0)

    def body(gr, ga, pr, pa, cf, out_ref):
        c = cf[0]
        rows = []
        for g, p in ((gr[0], pr[0]), (ga[0], pa[0])):
            d = p - g
            l = d * d * c
            m = g >= _THRESH
            rows.append(jnp.sum(jnp.where(m, one, zero)))
            rows.append(jnp.sum(jnp.where(m, l, zero)))
            rows.append(jnp.sum(l))
        block = jnp.concatenate(
            [jnp.full((1, 128), v, jnp.float32) for v in rows]
            + [jnp.zeros((2, 128), jnp.float32)], axis=0)
        out_ref[0] = block

    def run(gt_r, gt_a, p_r, p_a, cf):
        return pl.pallas_call(
            body,
            grid=(n,),
            in_specs=[pl.BlockSpec((1, H, W), lambda i: (i + first, 0, 0))
                      for _ in range(5)],
            out_specs=pl.BlockSpec((1, 8, 128), lambda i: (i, 0, 0)),
            out_shape=jax.ShapeDtypeStruct((n, 8, 128), jnp.float32),
        )(gt_r, gt_a, p_r, p_a, cf)

    return run


def _fb_body(P, gt_ref, pd_ref, cf_ref, out_ref):
    g = gt_ref[0, 0]
    p = pd_ref[0, 0]
    c = cf_ref[0]
    d = p - g
    l = d * d * c
    m = g >= _THRESH
    one = np.float32(1.0)
    zero = np.float32(0.0)
    npos = jnp.sum(jnp.where(m, one, zero))
    possum = jnp.sum(jnp.where(m, l, zero))
    negsum = jnp.sum(jnp.where(m, zero, l))
    nneg = np.float32(P) - npos
    k = np.float32(3.0) * npos
    neg_vals = jnp.where(m, _SENTINEL, l)

    def topsum(vals, kk):
        def bis(_, lohi):
            lo, hi = lohi
            mid = (lo + hi) * np.float32(0.5)
            cnt = jnp.sum(jnp.where(vals > mid, one, zero))
            take = cnt >= kk
            return (jnp.where(take, mid, lo), jnp.where(take, hi, mid))

        lo, hi = lax.fori_loop(
            0, 220, bis, (np.float32(-2.0), np.float32(2.0)))
        vk = hi
        gt_m = vals > vk
        cnt = jnp.sum(jnp.where(gt_m, one, zero))
        s = jnp.sum(jnp.where(gt_m, vals, zero))
        return s + (kk - cnt) * vk

    k_eff = jnp.clip(k, one, np.float32(P))
    topk_sum = topsum(neg_vals, k_eff)
    top500_sum = topsum(l, np.float32(500.0))

    pos_loss = possum / jnp.maximum(npos, one)
    neg_mean_all = negsum / jnp.maximum(nneg, one)
    topk_mean = topk_sum / jnp.maximum(k, one)
    negative_loss = jnp.where(nneg < k, neg_mean_all, topk_mean)
    contrib = jnp.where(
        npos > zero, pos_loss + negative_loss, top500_sum / np.float32(500.0))
    out_ref[0, 0, 0, :] = jnp.full((128,), contrib, jnp.float32)


def _make_fallback(B, H, W, interpret=False):
    P = H * W

    def fallback(gt_r, gt_a, p_r, p_a, cf):
        gt = jnp.stack([gt_r, gt_a])
        pd = jnp.stack([p_r, p_a])
        out = pl.pallas_call(
            functools.partial(_fb_body, P),
            grid=(2, B),
            in_specs=[
                pl.BlockSpec((1, 1, H, W), lambda t, s: (t, s, 0, 0)),
                pl.BlockSpec((1, 1, H, W), lambda t, s: (t, s, 0, 0)),
                pl.BlockSpec((1, H, W), lambda t, s: (s, 0, 0)),
            ],
            out_specs=pl.BlockSpec((1, 1, 1, 128), lambda t, s: (t, s, 0, 0)),
            out_shape=jax.ShapeDtypeStruct((2, B, 1, 128), jnp.float32),
            interpret=interpret,
        )(gt, pd, cf)
        return jnp.sum(out[:, :, 0, 0]) / np.float32(B)

    return fallback


def kernel(gt_region_scores, gt_affinity_scores, pred_region_scores,
           pred_affinity_scores, confidence_mask):
    B, H, W = gt_region_scores.shape
    P = H * W

    n_sc = B // 4
    sc_stats = _make_sc_stats(n_sc, H, W)
    tc_stats = _make_tc_stats(n_sc, B - n_sc, H, W)
    args = (gt_region_scores, gt_affinity_scores, pred_region_scores,
            pred_affinity_scores, confidence_mask)
    sc_out = sc_stats(*args)
    tc_out = tc_stats(*args)

    sub = 32 // n_sc
    st_sc = sc_out.reshape(n_sc, sub, 8, 128)[:, :, :6, :16].sum(axis=(1, 3))
    st_tc = tc_out[:, :6, 0]
    st = jnp.concatenate([st_sc, st_tc], axis=0)
    one = np.float32(1.0)
    cheap_total = np.float32(0.0)
    need = False
    for t in range(2):
        npos = st[:, 3 * t + 0]
        possum = st[:, 3 * t + 1]
        negsum = st[:, 3 * t + 2] - possum
        nneg = np.float32(P) - npos
        k = np.float32(3.0) * npos
        need = jnp.logical_or(need, jnp.any(
            jnp.logical_or(npos < np.float32(0.5), nneg >= k)))
        contrib = possum / jnp.maximum(npos, one) + negsum / jnp.maximum(nneg, one)
        cheap_total = cheap_total + jnp.sum(contrib)
    cheap = cheap_total / np.float32(B)

    fallback = _make_fallback(B, H, W)
    return lax.cond(
        need,
        lambda: fallback(gt_region_scores, gt_affinity_scores,
                         pred_region_scores, pred_affinity_scores,
                         confidence_mask),
        lambda: cheap,
    )
